# baseline (speedup 1.0000x reference)
import functools

import jax
import jax.numpy as jnp
from jax import lax
from jax.experimental import pallas as pl
from jax.experimental.pallas import tpu as pltpu
from jax.experimental.pallas import tpu_sc as plsc

_B = 4096
_L = 200
_D = 32
_N = _B * _L
_NC = 2
_NS = 16
_NW = _NC * _NS
_PER_W = _N // _NW
_CHUNK = 200
_GRP = 8
_NBUF = 2 * _GRP
_NGRP = _PER_W // (_CHUNK * _GRP)
_NPAIR = _NGRP // 2


def _build():
  mesh = plsc.VectorSubcoreMesh(core_axis_name="c", subcore_axis_name="s")

  @functools.partial(
      pl.kernel,
      mesh=mesh,
      out_type=jax.ShapeDtypeStruct((_N, _D), jnp.float32),
      scratch_types=[
          pltpu.VMEM((_NBUF, _CHUNK), jnp.int32),
          pltpu.VMEM((_NBUF, _CHUNK, _D), jnp.float32),
          pltpu.SemaphoreType.DMA((2,)),
          pltpu.SemaphoreType.DMA,
          pltpu.SemaphoreType.DMA((2,)),
      ],
      compiler_params=pltpu.CompilerParams(use_tc_tiling_on_sc=False),
  )
  def emb(idx_hbm, table_hbm, out_hbm, idx_v, rows_v, sem_i, sem_g, sem_s):
    wid = lax.axis_index("s") * _NC + lax.axis_index("c")
    base = wid * _PER_W

    @pl.loop(0, _NPAIR)
    def pair(p):
      for q in range(2):
        bs = q * _GRP
        g = p * 2 + q

        for b in range(_GRP):
          off = base + (g * _GRP + b) * _CHUNK
          pltpu.async_copy(
              idx_hbm.at[pl.ds(off, _CHUNK)], idx_v.at[bs + b], sem_i.at[q])

        @pl.when(p > 0)
        def _():
          for b in range(_GRP):
            pltpu.make_async_copy(
                rows_v.at[bs + b], out_hbm.at[pl.ds(base, _CHUNK)],
                sem_s.at[q]).wait()

        for b in range(_GRP):
          off = base + (g * _GRP + b) * _CHUNK
          pltpu.make_async_copy(
              idx_hbm.at[pl.ds(off, _CHUNK)], idx_v.at[bs + b],
              sem_i.at[q]).wait()
        gathers = []
        for b in range(_GRP):
          gathers.append(pltpu.async_copy(
              table_hbm.at[idx_v.at[bs + b]], rows_v.at[bs + b], sem_g))
        for cp in gathers:
          cp.wait()

        for b in range(_GRP):
          off = base + (g * _GRP + b) * _CHUNK
          pltpu.async_copy(
              rows_v.at[bs + b], out_hbm.at[pl.ds(off, _CHUNK)], sem_s.at[q])

    for q in range(2):
      for b in range(_GRP):
        pltpu.make_async_copy(
            rows_v.at[q * _GRP + b], out_hbm.at[pl.ds(base, _CHUNK)],
            sem_s.at[q]).wait()

  return emb


_emb = jax.jit(_build())


def kernel(x, table):
  idx = x.reshape(_N).astype(jnp.int32)
  out = _emb(idx, table)
  return out.reshape(_B, _L, _D)


# speedup vs baseline: 1.0014x; 1.0014x over previous
import functools

import jax
import jax.numpy as jnp
from jax import lax
from jax.experimental import pallas as pl
from jax.experimental.pallas import tpu as pltpu
from jax.experimental.pallas import tpu_sc as plsc

_B = 4096
_L = 200
_D = 32
_N = _B * _L
_NC = 2
_NS = 16
_NW = _NC * _NS
_PER_W = _N // _NW
_CHUNK = 400
_GRP = 4
_NBUF = 2 * _GRP
_NGRP = _PER_W // (_CHUNK * _GRP)
_NPAIR = _NGRP // 2


def _build():
  mesh = plsc.VectorSubcoreMesh(core_axis_name="c", subcore_axis_name="s")

  @functools.partial(
      pl.kernel,
      mesh=mesh,
      out_type=jax.ShapeDtypeStruct((_N, _D), jnp.float32),
      scratch_types=[
          pltpu.VMEM((_NBUF, _CHUNK), jnp.int32),
          pltpu.VMEM((_NBUF, _CHUNK, _D), jnp.float32),
          pltpu.SemaphoreType.DMA((2,)),
          pltpu.SemaphoreType.DMA,
          pltpu.SemaphoreType.DMA((2,)),
      ],
      compiler_params=pltpu.CompilerParams(use_tc_tiling_on_sc=False),
  )
  def emb(idx_hbm, table_hbm, out_hbm, idx_v, rows_v, sem_i, sem_g, sem_s):
    wid = lax.axis_index("s") * _NC + lax.axis_index("c")
    base = wid * _PER_W

    @pl.loop(0, _NPAIR)
    def pair(p):
      for q in range(2):
        bs = q * _GRP
        g = p * 2 + q

        for b in range(_GRP):
          off = base + (g * _GRP + b) * _CHUNK
          pltpu.async_copy(
              idx_hbm.at[pl.ds(off, _CHUNK)], idx_v.at[bs + b], sem_i.at[q])

        @pl.when(p > 0)
        def _():
          for b in range(_GRP):
            pltpu.make_async_copy(
                rows_v.at[bs + b], out_hbm.at[pl.ds(base, _CHUNK)],
                sem_s.at[q]).wait()

        for b in range(_GRP):
          off = base + (g * _GRP + b) * _CHUNK
          pltpu.make_async_copy(
              idx_hbm.at[pl.ds(off, _CHUNK)], idx_v.at[bs + b],
              sem_i.at[q]).wait()
        gathers = []
        for b in range(_GRP):
          gathers.append(pltpu.async_copy(
              table_hbm.at[idx_v.at[bs + b]], rows_v.at[bs + b], sem_g))
        for cp in gathers:
          cp.wait()

        for b in range(_GRP):
          off = base + (g * _GRP + b) * _CHUNK
          pltpu.async_copy(
              rows_v.at[bs + b], out_hbm.at[pl.ds(off, _CHUNK)], sem_s.at[q])

    for q in range(2):
      for b in range(_GRP):
        pltpu.make_async_copy(
            rows_v.at[q * _GRP + b], out_hbm.at[pl.ds(base, _CHUNK)],
            sem_s.at[q]).wait()

  return emb


_emb = jax.jit(_build())


def kernel(x, table):
  idx = x.reshape(_N).astype(jnp.int32)
  out = _emb(idx, table)
  return out.reshape(_B, _L, _D)
